# baseline (speedup 1.0000x reference)
import functools

import numpy as np
import jax
import jax.numpy as jnp
from jax import lax
from jax.experimental import pallas as pl
from jax.experimental.pallas import tpu as pltpu
from jax.experimental.pallas import tpu_sc as plsc

_NDIM = 64
_NC, _NS = 2, 16
_NW = _NC * _NS


def _dft_w():
    m = np.arange(_NDIM)
    mk = np.outer(m, m)
    c = np.cos(2.0 * np.pi * mk / _NDIM)
    s = np.sin(2.0 * np.pi * mk / _NDIM)
    return np.concatenate([c, s], axis=1).astype(np.float32)


_W = _dft_w()


def _sc_gather(entity_table, relation_table, s_idx, o_idx, r_idx):
    B = s_idx.shape[0]
    b_per_w = B // _NW
    mesh = plsc.VectorSubcoreMesh(core_axis_name="c", subcore_axis_name="s")

    @functools.partial(
        pl.kernel,
        mesh=mesh,
        compiler_params=pltpu.CompilerParams(use_tc_tiling_on_sc=False),
        out_type=(
            jax.ShapeDtypeStruct((B, _NDIM), jnp.float32),
            jax.ShapeDtypeStruct((B, _NDIM), jnp.float32),
            jax.ShapeDtypeStruct((B, _NDIM), jnp.float32),
        ),
        scratch_types=[
            pltpu.VMEM((b_per_w,), jnp.int32),
            pltpu.VMEM((b_per_w,), jnp.int32),
            pltpu.VMEM((b_per_w,), jnp.int32),
            pltpu.VMEM((b_per_w, _NDIM), jnp.float32),
            pltpu.VMEM((b_per_w, _NDIM), jnp.float32),
            pltpu.VMEM((b_per_w, _NDIM), jnp.float32),
            pltpu.SemaphoreType.DMA,
            pltpu.SemaphoreType.DMA,
            pltpu.SemaphoreType.DMA,
        ],
    )
    def k(ent_hbm, rel_hbm, s_hbm, o_hbm, r_hbm, es_out, eo_out, er_out,
          sidx_v, oidx_v, ridx_v, es_v, eo_v, er_v, sem_s, sem_o, sem_r):
        wid = lax.axis_index("s") * _NC + lax.axis_index("c")
        base = wid * b_per_w
        pltpu.sync_copy(s_hbm.at[pl.ds(base, b_per_w)], sidx_v)
        pltpu.sync_copy(o_hbm.at[pl.ds(base, b_per_w)], oidx_v)
        pltpu.sync_copy(r_hbm.at[pl.ds(base, b_per_w)], ridx_v)
        cp_s = pltpu.async_copy(ent_hbm.at[sidx_v], es_v, sem_s)
        cp_o = pltpu.async_copy(ent_hbm.at[oidx_v], eo_v, sem_o)
        cp_r = pltpu.async_copy(rel_hbm.at[ridx_v], er_v, sem_r)
        cp_s.wait()
        pltpu.sync_copy(es_v, es_out.at[pl.ds(base, b_per_w)])
        cp_o.wait()
        pltpu.sync_copy(eo_v, eo_out.at[pl.ds(base, b_per_w)])
        cp_r.wait()
        pltpu.sync_copy(er_v, er_out.at[pl.ds(base, b_per_w)])

    return k(entity_table, relation_table, s_idx, o_idx, r_idx)


def _score_body(w_ref, es_ref, eo_ref, er_ref, out_ref):
    w = w_ref[...]
    hp = lax.Precision.HIGHEST
    a = jnp.dot(es_ref[...], w, precision=hp, preferred_element_type=jnp.float32)
    b = jnp.dot(eo_ref[...], w, precision=hp, preferred_element_type=jnp.float32)
    r = jnp.dot(er_ref[...], w, precision=hp, preferred_element_type=jnp.float32)
    ac, as_ = a[:, :_NDIM], a[:, _NDIM:]
    bc, bs = b[:, :_NDIM], b[:, _NDIM:]
    rc, rs = r[:, :_NDIM], r[:, _NDIM:]
    eta = jnp.sum(rc * (ac * bc + as_ * bs) - rs * (as_ * bc - ac * bs),
                  axis=-1) * (1.0 / _NDIM)
    out_ref[...] = jax.nn.sigmoid(eta)


def _tc_score(es, eo, er, *, interpret=False):
    B = es.shape[0]
    blk = 2048
    grid = B // blk
    return pl.pallas_call(
        _score_body,
        grid=(grid,),
        in_specs=[
            pl.BlockSpec((_NDIM, 2 * _NDIM), lambda i: (0, 0)),
            pl.BlockSpec((blk, _NDIM), lambda i: (i, 0)),
            pl.BlockSpec((blk, _NDIM), lambda i: (i, 0)),
            pl.BlockSpec((blk, _NDIM), lambda i: (i, 0)),
        ],
        out_specs=pl.BlockSpec((blk,), lambda i: (i,)),
        out_shape=jax.ShapeDtypeStruct((B,), jnp.float32),
        interpret=interpret,
    )(jnp.asarray(_W), es, eo, er)


def kernel(entity_table, relation_table, s_idx, o_idx, r_idx):
    es, eo, er = _sc_gather(entity_table, relation_table, s_idx, o_idx, r_idx)
    return _tc_score(es, eo, er)


# speedup vs baseline: 1.9349x; 1.9349x over previous
import functools

import numpy as np
import jax
import jax.numpy as jnp
from jax import lax
from jax.experimental import pallas as pl
from jax.experimental.pallas import tpu as pltpu
from jax.experimental.pallas import tpu_sc as plsc

_NDIM = 64
_NC, _NS = 2, 16
_NW = _NC * _NS


def _dft_w():
    m = np.arange(_NDIM)
    mk = np.outer(m, m)
    c = np.cos(2.0 * np.pi * mk / _NDIM)
    s = np.sin(2.0 * np.pi * mk / _NDIM)
    return np.concatenate([c, s], axis=1).astype(np.float32)


_W = _dft_w()


_CHUNK = 8192
_HALF = _CHUNK // 2


def _relayout_body(in_ref, out_ref):
    x = in_ref[...]
    xt = x.T
    out_ref[:, :_NDIM] = xt[:_HALF, :]
    out_ref[:, _NDIM:] = xt[_HALF:, :]


def _tc_relayout(table_t, *, interpret=False):
    n = table_t.shape[1]
    grid = (n + _CHUNK - 1) // _CHUNK
    return pl.pallas_call(
        _relayout_body,
        grid=(grid,),
        in_specs=[pl.BlockSpec((_NDIM, _CHUNK), lambda i: (0, i))],
        out_specs=pl.BlockSpec((_HALF, 2 * _NDIM), lambda i: (i, 0)),
        out_shape=jax.ShapeDtypeStruct((grid * _HALF, 2 * _NDIM), jnp.float32),
        interpret=interpret,
    )(table_t)


def _sc_gather(ent_lin, relation_table, s_idx, o_idx, r_idx):
    B = s_idx.shape[0]
    b_per_w = B // _NW
    mesh = plsc.VectorSubcoreMesh(core_axis_name="c", subcore_axis_name="s")

    @functools.partial(
        pl.kernel,
        mesh=mesh,
        compiler_params=pltpu.CompilerParams(use_tc_tiling_on_sc=False),
        out_type=(
            jax.ShapeDtypeStruct((B, _NDIM), jnp.float32),
            jax.ShapeDtypeStruct((B, _NDIM), jnp.float32),
            jax.ShapeDtypeStruct((B, _NDIM), jnp.float32),
        ),
        scratch_types=[
            pltpu.VMEM((b_per_w,), jnp.int32),
            pltpu.VMEM((b_per_w,), jnp.int32),
            pltpu.VMEM((b_per_w,), jnp.int32),
            pltpu.VMEM((b_per_w, _NDIM), jnp.float32),
            pltpu.VMEM((b_per_w, _NDIM), jnp.float32),
            pltpu.VMEM((b_per_w, _NDIM), jnp.float32),
            pltpu.SemaphoreType.DMA,
            pltpu.SemaphoreType.DMA,
            pltpu.SemaphoreType.DMA,
        ],
    )
    def k(ent_hbm, rel_hbm, s_hbm, o_hbm, r_hbm, es_out, eo_out, er_out,
          sidx_v, oidx_v, ridx_v, es_v, eo_v, er_v, sem_s, sem_o, sem_r):
        wid = lax.axis_index("s") * _NC + lax.axis_index("c")
        base = wid * b_per_w
        pltpu.sync_copy(s_hbm.at[pl.ds(base, b_per_w)], sidx_v)
        pltpu.sync_copy(o_hbm.at[pl.ds(base, b_per_w)], oidx_v)
        pltpu.sync_copy(r_hbm.at[pl.ds(base, b_per_w)], ridx_v)
        for g in range(b_per_w // 16):
            sl = pl.ds(g * 16, 16)
            for ref in (sidx_v, oidx_v):
                iv = ref[sl]
                ref[sl] = (((iv >> 13) << 12) + (iv & 4095)) * 2 + ((iv >> 12) & 1)
        cp_s = pltpu.async_copy(ent_hbm.at[sidx_v], es_v, sem_s)
        cp_o = pltpu.async_copy(ent_hbm.at[oidx_v], eo_v, sem_o)
        cp_r = pltpu.async_copy(rel_hbm.at[ridx_v], er_v, sem_r)
        cp_s.wait()
        pltpu.sync_copy(es_v, es_out.at[pl.ds(base, b_per_w)])
        cp_o.wait()
        pltpu.sync_copy(eo_v, eo_out.at[pl.ds(base, b_per_w)])
        cp_r.wait()
        pltpu.sync_copy(er_v, er_out.at[pl.ds(base, b_per_w)])

    return k(ent_lin, relation_table, s_idx, o_idx, r_idx)


def _score_body(w_ref, es_ref, eo_ref, er_ref, out_ref):
    w = w_ref[...]
    hp = lax.Precision.HIGHEST
    a = jnp.dot(es_ref[...], w, precision=hp, preferred_element_type=jnp.float32)
    b = jnp.dot(eo_ref[...], w, precision=hp, preferred_element_type=jnp.float32)
    r = jnp.dot(er_ref[...], w, precision=hp, preferred_element_type=jnp.float32)
    ac, as_ = a[:, :_NDIM], a[:, _NDIM:]
    bc, bs = b[:, :_NDIM], b[:, _NDIM:]
    rc, rs = r[:, :_NDIM], r[:, _NDIM:]
    eta = jnp.sum(rc * (ac * bc + as_ * bs) - rs * (as_ * bc - ac * bs),
                  axis=-1) * (1.0 / _NDIM)
    out_ref[...] = jax.nn.sigmoid(eta)


def _tc_score(es, eo, er, *, interpret=False):
    B = es.shape[0]
    blk = 2048
    grid = B // blk
    return pl.pallas_call(
        _score_body,
        grid=(grid,),
        in_specs=[
            pl.BlockSpec((_NDIM, 2 * _NDIM), lambda i: (0, 0)),
            pl.BlockSpec((blk, _NDIM), lambda i: (i, 0)),
            pl.BlockSpec((blk, _NDIM), lambda i: (i, 0)),
            pl.BlockSpec((blk, _NDIM), lambda i: (i, 0)),
        ],
        out_specs=pl.BlockSpec((blk,), lambda i: (i,)),
        out_shape=jax.ShapeDtypeStruct((B,), jnp.float32),
        interpret=interpret,
    )(jnp.asarray(_W), es, eo, er)


def kernel(entity_table, relation_table, s_idx, o_idx, r_idx):
    ent_packed = _tc_relayout(entity_table.T)
    ent_lin = ent_packed.reshape(-1, _NDIM)
    es, eo, er = _sc_gather(ent_lin, relation_table, s_idx, o_idx, r_idx)
    return _tc_score(es, eo, er)


# speedup vs baseline: 2.1140x; 1.0926x over previous
import functools

import numpy as np
import jax
import jax.numpy as jnp
from jax import lax
from jax.experimental import pallas as pl
from jax.experimental.pallas import tpu as pltpu
from jax.experimental.pallas import tpu_sc as plsc

_NDIM = 64
_NC, _NS = 2, 16
_NW = _NC * _NS


def _dft_w():
    m = np.arange(_NDIM)
    mk = np.outer(m, m)
    c = np.cos(2.0 * np.pi * mk / _NDIM)
    s = np.sin(2.0 * np.pi * mk / _NDIM)
    return np.concatenate([c, s], axis=1).astype(np.float32)


_W = _dft_w()


_CHUNK = 16384
_LOG2C = 14
_HALF = _CHUNK // 2


def _relayout_body(in_ref, out_ref):
    x = in_ref[...]
    xt = x.T
    out_ref[:, :_NDIM] = xt[:_HALF, :]
    out_ref[:, _NDIM:] = xt[_HALF:, :]


def _tc_relayout(table_t, *, interpret=False):
    n = table_t.shape[1]
    grid = (n + _CHUNK - 1) // _CHUNK
    return pl.pallas_call(
        _relayout_body,
        grid=(grid,),
        in_specs=[pl.BlockSpec((_NDIM, _CHUNK), lambda i: (0, i))],
        out_specs=pl.BlockSpec((_HALF, 2 * _NDIM), lambda i: (i, 0)),
        out_shape=jax.ShapeDtypeStruct((grid * _HALF, 2 * _NDIM), jnp.float32),
        interpret=interpret,
    )(table_t)


def _sc_gather(ent_lin, relation_table, s_idx, o_idx, r_idx):
    B = s_idx.shape[0]
    b_per_w = B // _NW
    mesh = plsc.VectorSubcoreMesh(core_axis_name="c", subcore_axis_name="s")

    @functools.partial(
        pl.kernel,
        mesh=mesh,
        compiler_params=pltpu.CompilerParams(use_tc_tiling_on_sc=False),
        out_type=(
            jax.ShapeDtypeStruct((B, _NDIM), jnp.float32),
            jax.ShapeDtypeStruct((B, _NDIM), jnp.float32),
            jax.ShapeDtypeStruct((B, _NDIM), jnp.float32),
        ),
        scratch_types=[
            pltpu.VMEM((b_per_w,), jnp.int32),
            pltpu.VMEM((b_per_w,), jnp.int32),
            pltpu.VMEM((b_per_w,), jnp.int32),
            pltpu.VMEM((b_per_w, _NDIM), jnp.float32),
            pltpu.VMEM((b_per_w, _NDIM), jnp.float32),
            pltpu.VMEM((b_per_w, _NDIM), jnp.float32),
            pltpu.SemaphoreType.DMA,
            pltpu.SemaphoreType.DMA,
            pltpu.SemaphoreType.DMA,
        ],
    )
    def k(ent_hbm, rel_hbm, s_hbm, o_hbm, r_hbm, es_out, eo_out, er_out,
          sidx_v, oidx_v, ridx_v, es_v, eo_v, er_v, sem_s, sem_o, sem_r):
        wid = lax.axis_index("s") * _NC + lax.axis_index("c")
        base = wid * b_per_w
        pltpu.sync_copy(s_hbm.at[pl.ds(base, b_per_w)], sidx_v)
        pltpu.sync_copy(o_hbm.at[pl.ds(base, b_per_w)], oidx_v)
        pltpu.sync_copy(r_hbm.at[pl.ds(base, b_per_w)], ridx_v)
        for g in range(b_per_w // 16):
            sl = pl.ds(g * 16, 16)
            for ref in (sidx_v, oidx_v):
                iv = ref[sl]
                ref[sl] = ((((iv >> _LOG2C) << (_LOG2C - 1)) +
                            (iv & (_HALF - 1))) * 2 + ((iv >> (_LOG2C - 1)) & 1))
        cp_s = pltpu.async_copy(ent_hbm.at[sidx_v], es_v, sem_s)
        cp_o = pltpu.async_copy(ent_hbm.at[oidx_v], eo_v, sem_o)
        cp_r = pltpu.async_copy(rel_hbm.at[ridx_v], er_v, sem_r)
        cp_s.wait()
        pltpu.sync_copy(es_v, es_out.at[pl.ds(base, b_per_w)])
        cp_o.wait()
        pltpu.sync_copy(eo_v, eo_out.at[pl.ds(base, b_per_w)])
        cp_r.wait()
        pltpu.sync_copy(er_v, er_out.at[pl.ds(base, b_per_w)])

    return k(ent_lin, relation_table, s_idx, o_idx, r_idx)


def _score_body(w_ref, es_ref, eo_ref, er_ref, out_ref):
    w = w_ref[...]
    hp = lax.Precision.HIGHEST
    a = jnp.dot(es_ref[...], w, precision=hp, preferred_element_type=jnp.float32)
    b = jnp.dot(eo_ref[...], w, precision=hp, preferred_element_type=jnp.float32)
    r = jnp.dot(er_ref[...], w, precision=hp, preferred_element_type=jnp.float32)
    ac, as_ = a[:, :_NDIM], a[:, _NDIM:]
    bc, bs = b[:, :_NDIM], b[:, _NDIM:]
    rc, rs = r[:, :_NDIM], r[:, _NDIM:]
    eta = jnp.sum(rc * (ac * bc + as_ * bs) - rs * (as_ * bc - ac * bs),
                  axis=-1) * (1.0 / _NDIM)
    out_ref[...] = jax.nn.sigmoid(eta)


def _tc_score(es, eo, er, *, interpret=False):
    B = es.shape[0]
    blk = 2048
    grid = B // blk
    return pl.pallas_call(
        _score_body,
        grid=(grid,),
        in_specs=[
            pl.BlockSpec((_NDIM, 2 * _NDIM), lambda i: (0, 0)),
            pl.BlockSpec((blk, _NDIM), lambda i: (i, 0)),
            pl.BlockSpec((blk, _NDIM), lambda i: (i, 0)),
            pl.BlockSpec((blk, _NDIM), lambda i: (i, 0)),
        ],
        out_specs=pl.BlockSpec((blk,), lambda i: (i,)),
        out_shape=jax.ShapeDtypeStruct((B,), jnp.float32),
        interpret=interpret,
    )(jnp.asarray(_W), es, eo, er)


def kernel(entity_table, relation_table, s_idx, o_idx, r_idx):
    ent_packed = _tc_relayout(entity_table.T)
    ent_lin = ent_packed.reshape(-1, _NDIM)
    es, eo, er = _sc_gather(ent_lin, relation_table, s_idx, o_idx, r_idx)
    return _tc_score(es, eo, er)


# speedup vs baseline: 2.2065x; 1.0437x over previous
import functools

import numpy as np
import jax
import jax.numpy as jnp
from jax import lax
from jax.experimental import pallas as pl
from jax.experimental.pallas import tpu as pltpu
from jax.experimental.pallas import tpu_sc as plsc

_NDIM = 64
_NC, _NS = 2, 16
_NW = _NC * _NS


def _dft_w():
    m = np.arange(_NDIM)
    mk = np.outer(m, m)
    c = np.cos(2.0 * np.pi * mk / _NDIM)
    s = np.sin(2.0 * np.pi * mk / _NDIM)
    return np.concatenate([c, s], axis=1).astype(np.float32)


_W = _dft_w()


_CHUNK = 32768
_LOG2C = 15
_HALF = _CHUNK // 2


def _relayout_body(in_ref, out_ref):
    x = in_ref[...]
    xt = x.T
    out_ref[:, :_NDIM] = xt[:_HALF, :]
    out_ref[:, _NDIM:] = xt[_HALF:, :]


def _tc_relayout(table_t, *, interpret=False):
    n = table_t.shape[1]
    grid = (n + _CHUNK - 1) // _CHUNK
    return pl.pallas_call(
        _relayout_body,
        grid=(grid,),
        in_specs=[pl.BlockSpec((_NDIM, _CHUNK), lambda i: (0, i))],
        out_specs=pl.BlockSpec((_HALF, 2 * _NDIM), lambda i: (i, 0)),
        out_shape=jax.ShapeDtypeStruct((grid * _HALF, 2 * _NDIM), jnp.float32),
        interpret=interpret,
    )(table_t)


def _sc_gather(ent_lin, relation_table, s_idx, o_idx, r_idx):
    B = s_idx.shape[0]
    b_per_w = B // _NW
    mesh = plsc.VectorSubcoreMesh(core_axis_name="c", subcore_axis_name="s")

    @functools.partial(
        pl.kernel,
        mesh=mesh,
        compiler_params=pltpu.CompilerParams(use_tc_tiling_on_sc=False),
        out_type=(
            jax.ShapeDtypeStruct((B, _NDIM), jnp.float32),
            jax.ShapeDtypeStruct((B, _NDIM), jnp.float32),
            jax.ShapeDtypeStruct((B, _NDIM), jnp.float32),
        ),
        scratch_types=[
            pltpu.VMEM((b_per_w,), jnp.int32),
            pltpu.VMEM((b_per_w,), jnp.int32),
            pltpu.VMEM((b_per_w,), jnp.int32),
            pltpu.VMEM((b_per_w, _NDIM), jnp.float32),
            pltpu.VMEM((b_per_w, _NDIM), jnp.float32),
            pltpu.VMEM((b_per_w, _NDIM), jnp.float32),
            pltpu.SemaphoreType.DMA,
            pltpu.SemaphoreType.DMA,
            pltpu.SemaphoreType.DMA,
        ],
    )
    def k(ent_hbm, rel_hbm, s_hbm, o_hbm, r_hbm, es_out, eo_out, er_out,
          sidx_v, oidx_v, ridx_v, es_v, eo_v, er_v, sem_s, sem_o, sem_r):
        wid = lax.axis_index("s") * _NC + lax.axis_index("c")
        base = wid * b_per_w
        pltpu.sync_copy(s_hbm.at[pl.ds(base, b_per_w)], sidx_v)
        pltpu.sync_copy(o_hbm.at[pl.ds(base, b_per_w)], oidx_v)
        pltpu.sync_copy(r_hbm.at[pl.ds(base, b_per_w)], ridx_v)
        for g in range(b_per_w // 16):
            sl = pl.ds(g * 16, 16)
            for ref in (sidx_v, oidx_v):
                iv = ref[sl]
                ref[sl] = ((((iv >> _LOG2C) << (_LOG2C - 1)) +
                            (iv & (_HALF - 1))) * 2 + ((iv >> (_LOG2C - 1)) & 1))
        cp_s = pltpu.async_copy(ent_hbm.at[sidx_v], es_v, sem_s)
        cp_o = pltpu.async_copy(ent_hbm.at[oidx_v], eo_v, sem_o)
        cp_r = pltpu.async_copy(rel_hbm.at[ridx_v], er_v, sem_r)
        cp_s.wait()
        pltpu.sync_copy(es_v, es_out.at[pl.ds(base, b_per_w)])
        cp_o.wait()
        pltpu.sync_copy(eo_v, eo_out.at[pl.ds(base, b_per_w)])
        cp_r.wait()
        pltpu.sync_copy(er_v, er_out.at[pl.ds(base, b_per_w)])

    return k(ent_lin, relation_table, s_idx, o_idx, r_idx)


def _score_body(w_ref, es_ref, eo_ref, er_ref, out_ref):
    w = w_ref[...]
    hp = lax.Precision.HIGHEST
    a = jnp.dot(es_ref[...], w, precision=hp, preferred_element_type=jnp.float32)
    b = jnp.dot(eo_ref[...], w, precision=hp, preferred_element_type=jnp.float32)
    r = jnp.dot(er_ref[...], w, precision=hp, preferred_element_type=jnp.float32)
    ac, as_ = a[:, :_NDIM], a[:, _NDIM:]
    bc, bs = b[:, :_NDIM], b[:, _NDIM:]
    rc, rs = r[:, :_NDIM], r[:, _NDIM:]
    eta = jnp.sum(rc * (ac * bc + as_ * bs) - rs * (as_ * bc - ac * bs),
                  axis=-1) * (1.0 / _NDIM)
    out_ref[...] = jax.nn.sigmoid(eta)


def _tc_score(es, eo, er, *, interpret=False):
    B = es.shape[0]
    blk = 4096
    grid = B // blk
    return pl.pallas_call(
        _score_body,
        grid=(grid,),
        in_specs=[
            pl.BlockSpec((_NDIM, 2 * _NDIM), lambda i: (0, 0)),
            pl.BlockSpec((blk, _NDIM), lambda i: (i, 0)),
            pl.BlockSpec((blk, _NDIM), lambda i: (i, 0)),
            pl.BlockSpec((blk, _NDIM), lambda i: (i, 0)),
        ],
        out_specs=pl.BlockSpec((blk,), lambda i: (i,)),
        out_shape=jax.ShapeDtypeStruct((B,), jnp.float32),
        interpret=interpret,
    )(jnp.asarray(_W), es, eo, er)


def kernel(entity_table, relation_table, s_idx, o_idx, r_idx):
    ent_packed = _tc_relayout(entity_table.T)
    ent_lin = ent_packed.reshape(-1, _NDIM)
    es, eo, er = _sc_gather(ent_lin, relation_table, s_idx, o_idx, r_idx)
    return _tc_score(es, eo, er)
